# baseline (speedup 1.0000x reference)
import functools

import jax
import jax.numpy as jnp
from jax import lax
from jax.experimental import pallas as pl
from jax.experimental.pallas import tpu as pltpu
from jax.experimental.pallas import tpu_sc as plsc

N, E, D, R = 10000, 320000, 128, 16

E2 = E // 2
SCB = 3200
NSC = E2 // SCB
NG = SCB // 16
OWN = 625
REG = 640
ACC_R = 16 * REG
STG = SCB + 144
BN = 400
FB = 200


def _vgather(x, idx):
    dn = lax.GatherDimensionNumbers(
        offset_dims=(), collapsed_slice_dims=(0,), start_index_map=(0,))
    return lax.gather(x, idx[:, None], dn, slice_sizes=(1,),
                      mode=lax.GatherScatterMode.PROMISE_IN_BOUNDS)


def _hb_body(h_ref, gw_ref, gb_ref, bt_ref, hb_ref):
    h = h_ref[...]
    s = lax.dot_general(h, gw_ref[...], (((1,), (1,)), ((), ())),
                        preferred_element_type=jnp.float32)
    g = jax.nn.sigmoid(s + gb_ref[...])
    for r in range(R):
        hb_ref[r] = g[:, r:r + 1] * (h + bt_ref[r])


def _gidx_body(src_ref, et_ref, gidx_ref):
    gidx_ref[...] = et_ref[...] * N + src_ref[...]


def _final_body(h_ref, w_ref, p_ref, out_ref):
    out_ref[...] = (
        jnp.dot(h_ref[...], w_ref[...], preferred_element_type=jnp.float32)
        + p_ref[0] + p_ref[1])


def _sc_edge_body(hb, gidx, dst, part,
                  dstc_v, gidxc_v, stg_d, stg_g, rows_v, acc, sem):
    c = lax.axis_index("c")
    s = lax.axis_index("s")
    lo = s * OWN
    roff = s * (REG - OWN)
    zbase = s * REG
    trash = zbase + OWN

    zvec = jnp.zeros((16,), jnp.float32)

    def _zero(i, carry):
        for j in range(D // 16):
            rows_v[i, pl.ds(j * 16, 16)] = zvec
        return carry

    lax.fori_loop(0, 128, _zero, 0)
    for q in range(REG // 128):
        pltpu.sync_copy(rows_v, acc.at[pl.ds(zbase + q * 128, 128)])
    plsc.subcore_barrier()

    ebase = c * E2
    iota = lax.iota(jnp.int32, 16)
    pad_d = jnp.full((16,), trash, jnp.int32)
    pad_g = jnp.zeros((16,), jnp.int32)

    def _wave(w, carry):
        b = w * 128
        pltpu.async_copy(hb.at[stg_g.at[pl.ds(b, 128)]], rows_v, sem).wait()
        pltpu.sync_copy(rows_v, acc.at[stg_d.at[pl.ds(b, 128)]], add=True)
        return carry

    def _scan_chunk(k, carry):
        eb = ebase + k * SCB
        pltpu.sync_copy(dst.at[pl.ds(eb, SCB)], dstc_v)
        pltpu.sync_copy(gidx.at[pl.ds(eb, SCB)], gidxc_v)

        def _grp(j, cnt):
            sl = pl.ds(j * 16, 16)
            d = dstc_v[sl]
            gi = gidxc_v[sl]
            mask = (d >= lo) & (d < lo + OWN)
            x = jnp.where(mask, 1, 0)
            for dlt in (1, 2, 4, 8):
                sh = _vgather(x, jnp.maximum(iota - dlt, 0))
                x = x + jnp.where(iota >= dlt, sh, 0)
            perm = jnp.zeros((16,), jnp.int32)
            for jj in range(16):
                pj = _vgather(x, jnp.full((16,), jj, jnp.int32))
                perm = perm + jnp.where(pj <= iota, 1, 0)
            perm = jnp.minimum(perm, 15)
            stg_d[pl.ds(cnt, 16)] = _vgather(d, perm) + roff
            stg_g[pl.ds(cnt, 16)] = _vgather(gi, perm)
            return cnt + x[15]

        cnt = lax.fori_loop(0, NG, _grp, 0)
        for j in range(8):
            stg_d[pl.ds(cnt + j * 16, 16)] = pad_d
            stg_g[pl.ds(cnt + j * 16, 16)] = pad_g
        nw = (cnt + 127) // 128
        lax.fori_loop(0, nw, _wave, 0)
        return carry

    lax.fori_loop(0, NSC, _scan_chunk, 0)
    plsc.subcore_barrier()
    pltpu.sync_copy(acc.at[pl.ds(zbase, REG)], part.at[c, pl.ds(zbase, REG)])


@functools.cache
def _sc_call():
    return pl.kernel(
        _sc_edge_body,
        out_type=jax.ShapeDtypeStruct((2, ACC_R, D), jnp.float32),
        mesh=plsc.VectorSubcoreMesh(core_axis_name="c", subcore_axis_name="s"),
        scratch_types=[
            pltpu.VMEM((SCB,), jnp.int32),
            pltpu.VMEM((SCB,), jnp.int32),
            pltpu.VMEM((STG,), jnp.int32),
            pltpu.VMEM((STG,), jnp.int32),
            pltpu.VMEM((128, D), jnp.float32),
            pltpu.VMEM_SHARED((ACC_R, D), jnp.float32),
            pltpu.SemaphoreType.DMA,
        ],
    )


def kernel(h, edge_index, etypes, bias_term, gate_weight, gate_bias, loop_weight):
    src = edge_index[0].astype(jnp.int32)
    dst = edge_index[1].astype(jnp.int32)
    et = etypes.astype(jnp.int32)

    gw = gate_weight.reshape(R, D)
    gb = gate_bias.reshape(1, R)

    hb = pl.pallas_call(
        _hb_body,
        grid=(25,),
        in_specs=[
            pl.BlockSpec((BN, D), lambda i: (i, 0)),
            pl.BlockSpec((R, D), lambda i: (0, 0)),
            pl.BlockSpec((1, R), lambda i: (0, 0)),
            pl.BlockSpec((R, D), lambda i: (0, 0)),
        ],
        out_specs=pl.BlockSpec((R, BN, D), lambda i: (0, i, 0)),
        out_shape=jax.ShapeDtypeStruct((R, N, D), jnp.float32),
    )(h, gw, gb, bias_term)

    gidx2d = pl.pallas_call(
        _gidx_body,
        out_shape=jax.ShapeDtypeStruct((E // 128, 128), jnp.int32),
    )(src.reshape(E // 128, 128), et.reshape(E // 128, 128))

    part = _sc_call()(hb.reshape(R * N, D), gidx2d.reshape(E), dst)

    p = part.reshape(2, 16, REG, D)[:, :, :OWN, :].reshape(2, N, D)

    out = pl.pallas_call(
        _final_body,
        grid=(N // FB,),
        in_specs=[
            pl.BlockSpec((FB, D), lambda i: (i, 0)),
            pl.BlockSpec((D, D), lambda i: (0, 0)),
            pl.BlockSpec((2, FB, D), lambda i: (0, i, 0)),
        ],
        out_specs=pl.BlockSpec((FB, D), lambda i: (i, 0)),
        out_shape=jax.ShapeDtypeStruct((N, D), jnp.float32),
    )(h, loop_weight, p)
    return out


# speedup vs baseline: 6.9703x; 6.9703x over previous
import functools

import jax
import jax.numpy as jnp
from jax import lax
from jax.experimental import pallas as pl
from jax.experimental.pallas import tpu as pltpu
from jax.experimental.pallas import tpu_sc as plsc

N, E, D, R = 10000, 320000, 128, 16

E2 = E // 2
EPS = E2 // 16
CH = 2000
CHP = 2048
ZREG = 640
ACC_R = 16 * ZREG
TRASH = N + 64
BN = 400
FB = 200


def _vgather(x, idx):
    dn = lax.GatherDimensionNumbers(
        offset_dims=(), collapsed_slice_dims=(0,), start_index_map=(0,))
    return lax.gather(x, idx[:, None], dn, slice_sizes=(1,),
                      mode=lax.GatherScatterMode.PROMISE_IN_BOUNDS)


def _hb_body(h_ref, gw_ref, gb_ref, bt_ref, hb_ref):
    h = h_ref[...]
    s = lax.dot_general(h, gw_ref[...], (((1,), (1,)), ((), ())),
                        preferred_element_type=jnp.float32)
    g = jax.nn.sigmoid(s + gb_ref[...])
    for r in range(R):
        hb_ref[r] = g[:, r:r + 1] * (h + bt_ref[r])


def _gidx_body(src_ref, et_ref, gidx_ref):
    gidx_ref[...] = et_ref[...] * N + src_ref[...]


def _final_body(h_ref, w_ref, p_ref, out_ref):
    out_ref[...] = (
        jnp.dot(h_ref[...], w_ref[...], preferred_element_type=jnp.float32)
        + p_ref[0] + p_ref[1])


def _sc_edge_body(hb, gidx, dst, part, dstc_v, gidxc_v, rows_v, acc, sem):
    c = lax.axis_index("c")
    s = lax.axis_index("s")
    zbase = s * ZREG

    zvec = jnp.zeros((16,), jnp.float32)

    def _zero(i, carry):
        for j in range(D // 16):
            rows_v[i, pl.ds(j * 16, 16)] = zvec
        return carry

    lax.fori_loop(0, 128, _zero, 0)
    for q in range(ZREG // 128):
        pltpu.sync_copy(rows_v, acc.at[pl.ds(zbase + q * 128, 128)])
    plsc.subcore_barrier()

    ebase = c * E2 + s * EPS
    pad_d = jnp.full((16,), TRASH, jnp.int32)
    pad_g = jnp.zeros((16,), jnp.int32)

    def _wave(w, carry):
        b = w * 128
        pltpu.async_copy(hb.at[gidxc_v.at[pl.ds(b, 128)]], rows_v, sem).wait()
        pltpu.sync_copy(rows_v, acc.at[dstc_v.at[pl.ds(b, 128)]], add=True)
        return carry

    def _chunk(k, carry):
        eb = ebase + k * CH
        pltpu.sync_copy(dst.at[pl.ds(eb, CH)], dstc_v.at[pl.ds(0, CH)])
        pltpu.sync_copy(gidx.at[pl.ds(eb, CH)], gidxc_v.at[pl.ds(0, CH)])
        for j in range((CHP - CH) // 16):
            dstc_v[pl.ds(CH + j * 16, 16)] = pad_d
            gidxc_v[pl.ds(CH + j * 16, 16)] = pad_g
        lax.fori_loop(0, CHP // 128, _wave, 0)
        return carry

    lax.fori_loop(0, EPS // CH, _chunk, 0)
    plsc.subcore_barrier()
    pltpu.sync_copy(acc.at[pl.ds(zbase, ZREG)], part.at[c, pl.ds(zbase, ZREG)])


@functools.cache
def _sc_call():
    return pl.kernel(
        _sc_edge_body,
        out_type=jax.ShapeDtypeStruct((2, ACC_R, D), jnp.float32),
        mesh=plsc.VectorSubcoreMesh(core_axis_name="c", subcore_axis_name="s"),
        scratch_types=[
            pltpu.VMEM((CHP,), jnp.int32),
            pltpu.VMEM((CHP,), jnp.int32),
            pltpu.VMEM((128, D), jnp.float32),
            pltpu.VMEM_SHARED((ACC_R, D), jnp.float32),
            pltpu.SemaphoreType.DMA,
        ],
    )


def kernel(h, edge_index, etypes, bias_term, gate_weight, gate_bias, loop_weight):
    src = edge_index[0].astype(jnp.int32)
    dst = edge_index[1].astype(jnp.int32)
    et = etypes.astype(jnp.int32)

    gw = gate_weight.reshape(R, D)
    gb = gate_bias.reshape(1, R)

    hb = pl.pallas_call(
        _hb_body,
        grid=(25,),
        in_specs=[
            pl.BlockSpec((BN, D), lambda i: (i, 0)),
            pl.BlockSpec((R, D), lambda i: (0, 0)),
            pl.BlockSpec((1, R), lambda i: (0, 0)),
            pl.BlockSpec((R, D), lambda i: (0, 0)),
        ],
        out_specs=pl.BlockSpec((R, BN, D), lambda i: (0, i, 0)),
        out_shape=jax.ShapeDtypeStruct((R, N, D), jnp.float32),
    )(h, gw, gb, bias_term)

    gidx2d = pl.pallas_call(
        _gidx_body,
        out_shape=jax.ShapeDtypeStruct((E // 128, 128), jnp.int32),
    )(src.reshape(E // 128, 128), et.reshape(E // 128, 128))

    part = _sc_call()(hb.reshape(R * N, D), gidx2d.reshape(E), dst)

    p = part[:, :N, :]

    out = pl.pallas_call(
        _final_body,
        grid=(N // FB,),
        in_specs=[
            pl.BlockSpec((FB, D), lambda i: (i, 0)),
            pl.BlockSpec((D, D), lambda i: (0, 0)),
            pl.BlockSpec((2, FB, D), lambda i: (0, i, 0)),
        ],
        out_specs=pl.BlockSpec((FB, D), lambda i: (i, 0)),
        out_shape=jax.ShapeDtypeStruct((N, D), jnp.float32),
    )(h, loop_weight, p)
    return out


# speedup vs baseline: 7.4726x; 1.0721x over previous
import functools

import jax
import jax.numpy as jnp
from jax import lax
from jax.experimental import pallas as pl
from jax.experimental.pallas import tpu as pltpu
from jax.experimental.pallas import tpu_sc as plsc

N, E, D, R = 10000, 320000, 128, 16

E2 = E // 2
EPS = E2 // 16
CH = 2000
CHP = 2048
WV = 128
NW = CHP // WV
ZREG = 640
ACC_R = 16 * ZREG
TRASH = N + 64
BN = 400
FB = 200


def _vgather(x, idx):
    dn = lax.GatherDimensionNumbers(
        offset_dims=(), collapsed_slice_dims=(0,), start_index_map=(0,))
    return lax.gather(x, idx[:, None], dn, slice_sizes=(1,),
                      mode=lax.GatherScatterMode.PROMISE_IN_BOUNDS)


def _hb_body(h_ref, gw_ref, gb_ref, bt_ref, hb_ref):
    h = h_ref[...]
    s = lax.dot_general(h, gw_ref[...], (((1,), (1,)), ((), ())),
                        preferred_element_type=jnp.float32)
    g = jax.nn.sigmoid(s + gb_ref[...])
    for r in range(R):
        hb_ref[r] = g[:, r:r + 1] * (h + bt_ref[r])


def _gidx_body(src_ref, et_ref, gidx_ref):
    gidx_ref[...] = et_ref[...] * N + src_ref[...]


def _final_body(h_ref, w_ref, p_ref, out_ref):
    out_ref[...] = (
        jnp.dot(h_ref[...], w_ref[...], preferred_element_type=jnp.float32)
        + p_ref[0] + p_ref[1])


def _sc_edge_body(hb, gidx, dst, part,
                  dstc_v, gidxc_v, rows_a, rows_b, acc, sem_a, sem_b):
    c = lax.axis_index("c")
    s = lax.axis_index("s")
    zbase = s * ZREG

    zvec = jnp.zeros((16,), jnp.float32)

    def _zero(i, carry):
        for j in range(D // 16):
            rows_a[i, pl.ds(j * 16, 16)] = zvec
        return carry

    lax.fori_loop(0, WV, _zero, 0)
    for q in range(ZREG // 128):
        pltpu.sync_copy(rows_a.at[pl.ds(0, 128)],
                        acc.at[pl.ds(zbase + q * 128, 128)])
    plsc.subcore_barrier()

    ebase = c * E2 + s * EPS
    pad_d = jnp.full((16,), TRASH, jnp.int32)
    pad_g = jnp.zeros((16,), jnp.int32)
    bufs = (rows_a, rows_b)
    sems = (sem_a, sem_b)

    def _chunk(k, carry):
        eb = ebase + k * CH
        pltpu.sync_copy(dst.at[pl.ds(eb, CH)], dstc_v.at[pl.ds(0, CH)])
        pltpu.sync_copy(gidx.at[pl.ds(eb, CH)], gidxc_v.at[pl.ds(0, CH)])
        for j in range((CHP - CH) // 16):
            dstc_v[pl.ds(CH + j * 16, 16)] = pad_d
            gidxc_v[pl.ds(CH + j * 16, 16)] = pad_g

        def _gather(w):
            return pltpu.async_copy(
                hb.at[gidxc_v.at[pl.ds(w * WV, WV)]], bufs[w % 2], sems[w % 2])

        cps = [_gather(0), _gather(1)]
        for w in range(NW):
            cps[w % 2].wait()
            pltpu.sync_copy(bufs[w % 2],
                            acc.at[dstc_v.at[pl.ds(w * WV, WV)]], add=True)
            if w + 2 < NW:
                cps[w % 2] = _gather(w + 2)
        return carry

    lax.fori_loop(0, EPS // CH, _chunk, 0)
    plsc.subcore_barrier()
    pltpu.sync_copy(acc.at[pl.ds(zbase, ZREG)], part.at[c, pl.ds(zbase, ZREG)])


@functools.cache
def _sc_call():
    return pl.kernel(
        _sc_edge_body,
        out_type=jax.ShapeDtypeStruct((2, ACC_R, D), jnp.float32),
        mesh=plsc.VectorSubcoreMesh(core_axis_name="c", subcore_axis_name="s"),
        scratch_types=[
            pltpu.VMEM((CHP,), jnp.int32),
            pltpu.VMEM((CHP,), jnp.int32),
            pltpu.VMEM((WV, D), jnp.float32),
            pltpu.VMEM((WV, D), jnp.float32),
            pltpu.VMEM_SHARED((ACC_R, D), jnp.float32),
            pltpu.SemaphoreType.DMA,
            pltpu.SemaphoreType.DMA,
        ],
    )


def kernel(h, edge_index, etypes, bias_term, gate_weight, gate_bias, loop_weight):
    src = edge_index[0].astype(jnp.int32)
    dst = edge_index[1].astype(jnp.int32)
    et = etypes.astype(jnp.int32)

    gw = gate_weight.reshape(R, D)
    gb = gate_bias.reshape(1, R)

    hb = pl.pallas_call(
        _hb_body,
        grid=(25,),
        in_specs=[
            pl.BlockSpec((BN, D), lambda i: (i, 0)),
            pl.BlockSpec((R, D), lambda i: (0, 0)),
            pl.BlockSpec((1, R), lambda i: (0, 0)),
            pl.BlockSpec((R, D), lambda i: (0, 0)),
        ],
        out_specs=pl.BlockSpec((R, BN, D), lambda i: (0, i, 0)),
        out_shape=jax.ShapeDtypeStruct((R, N, D), jnp.float32),
    )(h, gw, gb, bias_term)

    gidx2d = pl.pallas_call(
        _gidx_body,
        out_shape=jax.ShapeDtypeStruct((E // 128, 128), jnp.int32),
    )(src.reshape(E // 128, 128), et.reshape(E // 128, 128))

    part = _sc_call()(hb.reshape(R * N, D), gidx2d.reshape(E), dst)

    p = part[:, :N, :]

    out = pl.pallas_call(
        _final_body,
        grid=(N // FB,),
        in_specs=[
            pl.BlockSpec((FB, D), lambda i: (i, 0)),
            pl.BlockSpec((D, D), lambda i: (0, 0)),
            pl.BlockSpec((2, FB, D), lambda i: (0, i, 0)),
        ],
        out_specs=pl.BlockSpec((FB, D), lambda i: (i, 0)),
        out_shape=jax.ShapeDtypeStruct((N, D), jnp.float32),
    )(h, loop_weight, p)
    return out


# speedup vs baseline: 7.4779x; 1.0007x over previous
import functools

import jax
import jax.numpy as jnp
from jax import lax
from jax.experimental import pallas as pl
from jax.experimental.pallas import tpu as pltpu
from jax.experimental.pallas import tpu_sc as plsc

N, E, D, R = 10000, 320000, 128, 16

E2 = E // 2
EPS = E2 // 16
CH = 2000
CHP = 2048
WV = 128
NW = CHP // WV
ZREG = 640
ACC_R = 16 * ZREG
TRASH = N + 64
BN = 400
FB = 200


def _hb_body(h_ref, gw_ref, gb_ref, bt_ref, hb_ref):
    h = h_ref[...]
    s = lax.dot_general(h, gw_ref[...], (((1,), (1,)), ((), ())),
                        preferred_element_type=jnp.float32)
    g = jax.nn.sigmoid(s + gb_ref[...])
    for r in range(R):
        hb_ref[r] = g[:, r:r + 1] * (h + bt_ref[r])


def _gidx_body(src_ref, et_ref, gidx_ref):
    gidx_ref[...] = et_ref[...] * N + src_ref[...]


def _final_body(h_ref, w_ref, p_ref, out_ref):
    out_ref[...] = (
        jnp.dot(h_ref[...], w_ref[...], preferred_element_type=jnp.float32)
        + p_ref[0] + p_ref[1])


def _sc_edge_body(hb, gidx, dst, part,
                  dstc_v, gidxc_v, rows_a, rows_b, acc, sem_a, sem_b):
    c = lax.axis_index("c")
    s = lax.axis_index("s")
    zbase = s * ZREG

    zvec = jnp.zeros((16,), jnp.float32)

    def _zero(i, carry):
        for j in range(D // 16):
            rows_a[i, pl.ds(j * 16, 16)] = zvec
        return carry

    lax.fori_loop(0, WV, _zero, 0)
    for q in range(ZREG // 128):
        pltpu.sync_copy(rows_a.at[pl.ds(0, 128)],
                        acc.at[pl.ds(zbase + q * 128, 128)])
    plsc.subcore_barrier()

    ebase = c * E2 + s * EPS
    pad_d = jnp.full((16,), TRASH, jnp.int32)
    pad_g = jnp.zeros((16,), jnp.int32)
    bufs = (rows_a, rows_b)
    sems = (sem_a, sem_b)

    def _chunk(k, carry):
        eb = ebase + k * CH
        pltpu.sync_copy(dst.at[pl.ds(eb, CH)], dstc_v.at[pl.ds(0, CH)])
        pltpu.sync_copy(gidx.at[pl.ds(eb, CH)], gidxc_v.at[pl.ds(0, CH)])
        for j in range((CHP - CH) // 16):
            dstc_v[pl.ds(CH + j * 16, 16)] = pad_d
            gidxc_v[pl.ds(CH + j * 16, 16)] = pad_g

        def _gather(w):
            return pltpu.async_copy(
                hb.at[gidxc_v.at[pl.ds(w * WV, WV)]], bufs[w % 2], sems[w % 2])

        cps = [_gather(0), _gather(1)]
        for w in range(NW):
            cps[w % 2].wait()
            pltpu.sync_copy(bufs[w % 2],
                            acc.at[dstc_v.at[pl.ds(w * WV, WV)]], add=True)
            if w + 2 < NW:
                cps[w % 2] = _gather(w + 2)
        return carry

    lax.fori_loop(0, EPS // CH, _chunk, 0)
    plsc.subcore_barrier()
    pltpu.sync_copy(acc.at[pl.ds(zbase, ZREG)], part.at[c, pl.ds(zbase, ZREG)])


@functools.cache
def _sc_call():
    return pl.kernel(
        _sc_edge_body,
        out_type=jax.ShapeDtypeStruct((2, ACC_R, D), jnp.float32),
        mesh=plsc.VectorSubcoreMesh(core_axis_name="c", subcore_axis_name="s"),
        scratch_types=[
            pltpu.VMEM((CHP,), jnp.int32),
            pltpu.VMEM((CHP,), jnp.int32),
            pltpu.VMEM((WV, D), jnp.float32),
            pltpu.VMEM((WV, D), jnp.float32),
            pltpu.VMEM_SHARED((ACC_R, D), jnp.float32),
            pltpu.SemaphoreType.DMA,
            pltpu.SemaphoreType.DMA,
        ],
    )


def kernel(h, edge_index, etypes, bias_term, gate_weight, gate_bias, loop_weight):
    src = edge_index[0].astype(jnp.int32)
    dst = edge_index[1].astype(jnp.int32)
    et = etypes.astype(jnp.int32)

    gw = gate_weight.reshape(R, D)
    gb = gate_bias.reshape(1, R)

    hb = pl.pallas_call(
        _hb_body,
        grid=(25,),
        in_specs=[
            pl.BlockSpec((BN, D), lambda i: (i, 0)),
            pl.BlockSpec((R, D), lambda i: (0, 0)),
            pl.BlockSpec((1, R), lambda i: (0, 0)),
            pl.BlockSpec((R, D), lambda i: (0, 0)),
        ],
        out_specs=pl.BlockSpec((R, BN, D), lambda i: (0, i, 0)),
        out_shape=jax.ShapeDtypeStruct((R, N, D), jnp.float32),
    )(h, gw, gb, bias_term)

    gidx2d = pl.pallas_call(
        _gidx_body,
        out_shape=jax.ShapeDtypeStruct((E // 128, 128), jnp.int32),
    )(src.reshape(E // 128, 128), et.reshape(E // 128, 128))

    part = _sc_call()(hb.reshape(R * N, D), gidx2d.reshape(E), dst)

    p = part[:, :N, :]

    out = pl.pallas_call(
        _final_body,
        grid=(N // FB,),
        in_specs=[
            pl.BlockSpec((FB, D), lambda i: (i, 0)),
            pl.BlockSpec((D, D), lambda i: (0, 0)),
            pl.BlockSpec((2, FB, D), lambda i: (0, i, 0)),
        ],
        out_specs=pl.BlockSpec((FB, D), lambda i: (i, 0)),
        out_shape=jax.ShapeDtypeStruct((N, D), jnp.float32),
    )(h, loop_weight, p)
    return out
